# Initial kernel scaffold; baseline (speedup 1.0000x reference)
import jax
import jax.numpy as jnp
from jax.experimental import pallas as pl


def kernel(x, edge_index, edge_type, w0, root0, b0, w1, root1, b1, w2, root2, b2):
    raise NotImplementedError("write your pallas kernel here")


# speedup vs baseline: 8.1675x; 8.1675x over previous
import functools

import jax
import jax.numpy as jnp
from jax import lax
from jax.experimental import pallas as pl
from jax.experimental.pallas import tpu as pltpu
from jax.experimental.pallas import tpu_sc as plsc

N = 10000
E = 320000
R = 8
H = 128
NC = 2
NS = 16
LANES = 16
NCH = H // LANES
CPS = NCH // NC
BE = 128
_QUANT = NC * NS * BE * 8
E_PAD = ((E + _QUANT - 1) // _QUANT) * _QUANT
NB = E_PAD // BE
NB_T = NB // NS
NB_TC = NB // (NC * NS)
ROWS = R * N
ZROWS = 1264
ROWS_PAD = NS * 4 * ZROWS
WB = ROWS // NS
TN = 1000

_mesh = plsc.VectorSubcoreMesh(core_axis_name="c", subcore_axis_name="s")
_sc_params = pltpu.CompilerParams(use_tc_tiling_on_sc=False)


def _zero_acc(zeros_hbm, acc, s):
  @pl.loop(0, 4)
  def _(z):
    pltpu.sync_copy(zeros_hbm, acc.at[pl.ds((s * 4 + z) * ZROWS, ZROWS), :])


@functools.partial(
    pl.kernel,
    out_type=jax.ShapeDtypeStruct((NC, ROWS, LANES), jnp.float32),
    mesh=_mesh,
    compiler_params=_sc_params,
    scratch_types=[
        pltpu.VMEM_SHARED((ROWS_PAD, LANES), jnp.float32),
        pltpu.VMEM((NB_TC, BE), jnp.int32),
        pltpu.VMEM((BE, LANES), jnp.float32),
    ],
)
def _sc_counts(jidx_hbm, zeros_hbm, ones_hbm, cnt_hbm, acc, jidx_v, ones_v):
  c = lax.axis_index("c")
  s = lax.axis_index("s")
  pltpu.sync_copy(ones_hbm, ones_v)
  _zero_acc(zeros_hbm, acc, s)
  base = (c * NS + s) * NB_TC
  pltpu.sync_copy(jidx_hbm.at[pl.ds(base, NB_TC), :], jidx_v)
  plsc.subcore_barrier()

  @pl.loop(0, NB_TC)
  def _(b):
    pltpu.sync_copy(ones_v, acc.at[jidx_v.at[b]], add=True)

  plsc.subcore_barrier()
  pltpu.sync_copy(acc.at[pl.ds(s * WB, WB), :],
                  cnt_hbm.at[c, pl.ds(s * WB, WB), :])


@functools.partial(
    pl.kernel,
    out_type=jax.ShapeDtypeStruct((ROWS, H), jnp.float32),
    mesh=_mesh,
    compiler_params=_sc_params,
    scratch_types=[
        pltpu.VMEM_SHARED((ROWS_PAD, LANES), jnp.float32),
        pltpu.VMEM((NB_T, BE), jnp.int32),
        pltpu.VMEM((NB_T, BE), jnp.int32),
        pltpu.VMEM((BE, LANES), jnp.float32),
    ],
)
def _sc_scatter(xflat_hbm, gidx_hbm, jidx_hbm, zeros_hbm, s_hbm, acc, gidx_v,
                jidx_v, rows_v):
  c = lax.axis_index("c")
  s = lax.axis_index("s")
  pltpu.sync_copy(jidx_hbm.at[pl.ds(s * NB_T, NB_T), :], jidx_v)

  @pl.loop(0, CPS)
  def _(ci):
    ch = c * CPS + ci
    _zero_acc(zeros_hbm, acc, s)
    pltpu.sync_copy(gidx_hbm.at[ch, pl.ds(s * NB_T, NB_T), :], gidx_v)
    plsc.subcore_barrier()

    @pl.loop(0, NB_T)
    def _(b):
      pltpu.sync_copy(xflat_hbm.at[gidx_v.at[b]], rows_v)
      pltpu.sync_copy(rows_v, acc.at[jidx_v.at[b]], add=True)

    plsc.subcore_barrier()
    pltpu.sync_copy(acc.at[pl.ds(s * WB, WB), :],
                    s_hbm.at[pl.ds(s * WB, WB), pl.ds(ch * LANES, LANES)])
    plsc.subcore_barrier()


def _combine_body(relu, s_ref, inv_ref, x_ref, w_ref, root_ref, b_ref, o_ref):
  acc = jnp.dot(x_ref[...], root_ref[...], preferred_element_type=jnp.float32)
  acc = acc + b_ref[...]
  for r in range(R):
    sr = s_ref[r] * inv_ref[:, r:r + 1]
    acc = acc + jnp.dot(sr, w_ref[r], preferred_element_type=jnp.float32)
  if relu:
    acc = jnp.maximum(acc, 0.0)
  o_ref[...] = acc


def _combine(s, inv_t, x, w, root, bias, relu):
  return pl.pallas_call(
      functools.partial(_combine_body, relu),
      out_shape=jax.ShapeDtypeStruct((N, H), jnp.float32),
      grid=(N // TN,),
      in_specs=[
          pl.BlockSpec((R, TN, H), lambda i: (0, i, 0)),
          pl.BlockSpec((TN, R), lambda i: (i, 0)),
          pl.BlockSpec((TN, H), lambda i: (i, 0)),
          pl.BlockSpec((R, H, H), lambda i: (0, 0, 0)),
          pl.BlockSpec((H, H), lambda i: (0, 0)),
          pl.BlockSpec((1, H), lambda i: (0, 0)),
      ],
      out_specs=pl.BlockSpec((TN, H), lambda i: (i, 0)),
  )(s.reshape(R, N, H), inv_t, x, w, root, bias.reshape(1, H))


def _expand_blockdiag(w):
  nb, bi, bo = w.shape[1], w.shape[2], w.shape[3]
  wd = jnp.zeros((R, nb * bi, nb * bo), jnp.float32)
  for b in range(nb):
    wd = wd.at[:, b * bi:(b + 1) * bi, b * bo:(b + 1) * bo].set(w[:, b])
  return wd


def kernel(x, edge_index, edge_type, w0, root0, b0, w1, root1, b1, w2, root2,
           b2):
  src = edge_index[0].astype(jnp.int32)
  dst = edge_index[1].astype(jnp.int32)
  et = edge_type.astype(jnp.int32)
  pad = E_PAD - E
  gsrc = jnp.concatenate([src, jnp.zeros((pad,), jnp.int32)])
  gidx = (gsrc[None, :] * NCH +
          jnp.arange(NCH, dtype=jnp.int32)[:, None]).reshape(NCH, NB, BE)
  jidx = jnp.concatenate(
      [et * N + dst, jnp.full((pad,), ROWS, jnp.int32)]).reshape(NB, BE)
  zeros = jnp.zeros((ZROWS, LANES), jnp.float32)
  ones = jnp.ones((BE, LANES), jnp.float32)

  cnt = _sc_counts(jidx, zeros, ones)
  csum = cnt[0, :, 0] + cnt[1, :, 0]
  inv_t = (1.0 / jnp.clip(csum, 1.0)).reshape(R, N).T

  wd0 = _expand_blockdiag(w0)
  wd1 = _expand_blockdiag(w1)

  h = x
  for (w, root, b, relu) in ((wd0, root0, b0, True), (wd1, root1, b1, True),
                             (w2, root2, b2, False)):
    s = _sc_scatter(h.reshape(N * NCH, LANES), gidx, jidx, zeros)
    h = _combine(s, inv_t, h, w, root, b, relu)
  return h


# speedup vs baseline: 10.4690x; 1.2818x over previous
import functools

import jax
import jax.numpy as jnp
from jax import lax
from jax.experimental import pallas as pl
from jax.experimental.pallas import tpu as pltpu
from jax.experimental.pallas import tpu_sc as plsc

N = 10000
E = 320000
R = 8
H = 128
NC = 2
NS = 16
LANES = 16
NCH = H // LANES
CPS = NCH // NC
BE = 128
_QUANT = NC * NS * BE * 8
E_PAD = ((E + _QUANT - 1) // _QUANT) * _QUANT
NB = E_PAD // BE
NB_T = NB // NS
NB_TC = NB // (NC * NS)
ROWS = R * N
ZROWS = 1264
ROWS_PAD = NS * 4 * ZROWS
WB = ROWS // NS
KG = 8
NG = NB_T // KG
EPT = E_PAD // NS
TN = 1000

_mesh = plsc.VectorSubcoreMesh(core_axis_name="c", subcore_axis_name="s")
_sc_params = pltpu.CompilerParams(use_tc_tiling_on_sc=False)


def _zero_acc(zeros_hbm, acc, s):
  @pl.loop(0, 4)
  def _(z):
    pltpu.sync_copy(zeros_hbm, acc.at[pl.ds((s * 4 + z) * ZROWS, ZROWS), :])


@functools.partial(
    pl.kernel,
    out_type=jax.ShapeDtypeStruct((NC, ROWS, LANES), jnp.float32),
    mesh=_mesh,
    compiler_params=_sc_params,
    scratch_types=[
        pltpu.VMEM_SHARED((ROWS_PAD, LANES), jnp.float32),
        pltpu.VMEM((KG * BE,), jnp.int32),
        pltpu.VMEM((KG * BE, LANES), jnp.float32),
    ],
)
def _sc_counts(jidx_hbm, zeros_hbm, ones_hbm, cnt_hbm, acc, jidx_v, ones_v):
  c = lax.axis_index("c")
  s = lax.axis_index("s")
  pltpu.sync_copy(ones_hbm, ones_v)
  _zero_acc(zeros_hbm, acc, s)
  plsc.subcore_barrier()

  @pl.loop(0, NG // 2)
  def _(g):
    base = (c * NS + s) * (EPT // 2) + g * (KG * BE)
    pltpu.sync_copy(jidx_hbm.at[pl.ds(base, KG * BE)], jidx_v)
    pltpu.sync_copy(ones_v, acc.at[jidx_v], add=True)

  plsc.subcore_barrier()
  pltpu.sync_copy(acc.at[pl.ds(s * WB, WB), :],
                  cnt_hbm.at[c, pl.ds(s * WB, WB), :])


@functools.partial(
    pl.kernel,
    out_type=jax.ShapeDtypeStruct((ROWS, H), jnp.float32),
    mesh=_mesh,
    compiler_params=_sc_params,
    scratch_types=[
        pltpu.VMEM_SHARED((ROWS_PAD, LANES), jnp.float32),
        pltpu.VMEM((KG * BE,), jnp.int32),
        pltpu.VMEM((KG * BE,), jnp.int32),
        pltpu.VMEM((KG * BE, LANES), jnp.float32),
    ],
)
def _sc_scatter(xflat_hbm, gidx_hbm, jidx_hbm, zeros_hbm, s_hbm, acc, gidx_v,
                jidx_v, rows_v):
  c = lax.axis_index("c")
  s = lax.axis_index("s")

  @pl.loop(0, CPS)
  def _(ci):
    ch = c * CPS + ci
    _zero_acc(zeros_hbm, acc, s)
    plsc.subcore_barrier()

    @pl.loop(0, NG)
    def _(g):
      base = s * EPT + g * (KG * BE)
      pltpu.sync_copy(gidx_hbm.at[ch, pl.ds(base, KG * BE)], gidx_v)
      pltpu.sync_copy(jidx_hbm.at[pl.ds(base, KG * BE)], jidx_v)
      pltpu.sync_copy(xflat_hbm.at[gidx_v], rows_v)
      pltpu.sync_copy(rows_v, acc.at[jidx_v], add=True)

    plsc.subcore_barrier()
    pltpu.sync_copy(acc.at[pl.ds(s * WB, WB), :],
                    s_hbm.at[pl.ds(s * WB, WB), pl.ds(ch * LANES, LANES)])
    plsc.subcore_barrier()


def _combine_body(relu, s_ref, inv_ref, x_ref, w_ref, root_ref, b_ref, o_ref):
  acc = jnp.dot(x_ref[...], root_ref[...], preferred_element_type=jnp.float32)
  acc = acc + b_ref[...]
  for r in range(R):
    sr = s_ref[r] * inv_ref[:, r:r + 1]
    acc = acc + jnp.dot(sr, w_ref[r], preferred_element_type=jnp.float32)
  if relu:
    acc = jnp.maximum(acc, 0.0)
  o_ref[...] = acc


def _combine(s, inv_t, x, w, root, bias, relu):
  return pl.pallas_call(
      functools.partial(_combine_body, relu),
      out_shape=jax.ShapeDtypeStruct((N, H), jnp.float32),
      grid=(N // TN,),
      in_specs=[
          pl.BlockSpec((R, TN, H), lambda i: (0, i, 0)),
          pl.BlockSpec((TN, R), lambda i: (i, 0)),
          pl.BlockSpec((TN, H), lambda i: (i, 0)),
          pl.BlockSpec((R, H, H), lambda i: (0, 0, 0)),
          pl.BlockSpec((H, H), lambda i: (0, 0)),
          pl.BlockSpec((1, H), lambda i: (0, 0)),
      ],
      out_specs=pl.BlockSpec((TN, H), lambda i: (i, 0)),
  )(s.reshape(R, N, H), inv_t, x, w, root, bias.reshape(1, H))


def _expand_blockdiag(w):
  nb, bi, bo = w.shape[1], w.shape[2], w.shape[3]
  wd = jnp.zeros((R, nb * bi, nb * bo), jnp.float32)
  for b in range(nb):
    wd = wd.at[:, b * bi:(b + 1) * bi, b * bo:(b + 1) * bo].set(w[:, b])
  return wd


def kernel(x, edge_index, edge_type, w0, root0, b0, w1, root1, b1, w2, root2,
           b2):
  src = edge_index[0].astype(jnp.int32)
  dst = edge_index[1].astype(jnp.int32)
  et = edge_type.astype(jnp.int32)
  pad = E_PAD - E
  gsrc = jnp.concatenate([src, jnp.zeros((pad,), jnp.int32)])
  gidx = (gsrc[None, :] * NCH +
          jnp.arange(NCH, dtype=jnp.int32)[:, None])
  jidx = jnp.concatenate([et * N + dst, jnp.full((pad,), ROWS, jnp.int32)])
  zeros = jnp.zeros((ZROWS, LANES), jnp.float32)
  ones = jnp.ones((KG * BE, LANES), jnp.float32)

  cnt = _sc_counts(jidx, zeros, ones)
  csum = cnt[0, :, 0] + cnt[1, :, 0]
  inv_t = (1.0 / jnp.clip(csum, 1.0)).reshape(R, N).T

  wd0 = _expand_blockdiag(w0)
  wd1 = _expand_blockdiag(w1)

  h = x
  for (w, root, b, relu) in ((wd0, root0, b0, True), (wd1, root1, b1, True),
                             (w2, root2, b2, False)):
    s = _sc_scatter(h.reshape(N * NCH, LANES), gidx, jidx, zeros)
    h = _combine(s, inv_t, h, w, root, b, relu)
  return h


# speedup vs baseline: 11.3950x; 1.0884x over previous
import functools

import jax
import jax.numpy as jnp
from jax import lax
from jax.experimental import pallas as pl
from jax.experimental.pallas import tpu as pltpu
from jax.experimental.pallas import tpu_sc as plsc

N = 10000
E = 320000
R = 8
H = 128
NC = 2
NS = 16
LANES = 16
NCH = H // LANES
CPS = NCH // NC
BE = 128
_QUANT = NC * NS * BE * 8
E_PAD = ((E + _QUANT - 1) // _QUANT) * _QUANT
NB = E_PAD // BE
NB_T = NB // NS
NB_TC = NB // (NC * NS)
ROWS = R * N
ZROWS = 1264
ROWS_PAD = NS * 4 * ZROWS
WB = ROWS // NS
KG = 8
NG = NB_T // KG
EPT = E_PAD // NS
TN = 1000

_mesh = plsc.VectorSubcoreMesh(core_axis_name="c", subcore_axis_name="s")
_sc_params = pltpu.CompilerParams(use_tc_tiling_on_sc=False)


def _zero_acc(zeros_hbm, acc, s):
  @pl.loop(0, 4)
  def _(z):
    pltpu.sync_copy(zeros_hbm, acc.at[pl.ds((s * 4 + z) * ZROWS, ZROWS), :])


@functools.partial(
    pl.kernel,
    out_type=jax.ShapeDtypeStruct((NC, ROWS, LANES), jnp.float32),
    mesh=_mesh,
    compiler_params=_sc_params,
    scratch_types=[
        pltpu.VMEM_SHARED((ROWS_PAD, LANES), jnp.float32),
        pltpu.VMEM((KG * BE,), jnp.int32),
        pltpu.VMEM((KG * BE, LANES), jnp.float32),
    ],
)
def _sc_counts(jidx_hbm, zeros_hbm, ones_hbm, cnt_hbm, acc, jidx_v, ones_v):
  c = lax.axis_index("c")
  s = lax.axis_index("s")
  pltpu.sync_copy(ones_hbm, ones_v)
  _zero_acc(zeros_hbm, acc, s)
  plsc.subcore_barrier()

  @pl.loop(0, NG // 2)
  def _(g):
    base = (c * NS + s) * (EPT // 2) + g * (KG * BE)
    pltpu.sync_copy(jidx_hbm.at[pl.ds(base, KG * BE)], jidx_v)
    pltpu.sync_copy(ones_v, acc.at[jidx_v], add=True)

  plsc.subcore_barrier()
  pltpu.sync_copy(acc.at[pl.ds(s * WB, WB), :],
                  cnt_hbm.at[c, pl.ds(s * WB, WB), :])


@functools.partial(
    pl.kernel,
    out_type=jax.ShapeDtypeStruct((ROWS, H), jnp.float32),
    mesh=_mesh,
    compiler_params=_sc_params,
    scratch_types=[
        pltpu.VMEM_SHARED((ROWS_PAD, LANES), jnp.float32),
        pltpu.VMEM((KG * BE,), jnp.int32),
        pltpu.VMEM((KG * BE,), jnp.int32),
        pltpu.VMEM((KG * BE,), jnp.int32),
        pltpu.VMEM((KG * BE,), jnp.int32),
        pltpu.VMEM((KG * BE, LANES), jnp.float32),
        pltpu.VMEM((KG * BE, LANES), jnp.float32),
        pltpu.SemaphoreType.DMA,
        pltpu.SemaphoreType.DMA,
        pltpu.SemaphoreType.DMA,
        pltpu.SemaphoreType.DMA,
    ],
)
def _sc_scatter(xflat_hbm, gidx_hbm, jidx_hbm, zeros_hbm, s_hbm, acc, g0, g1,
                j0, j1, r0, r1, sg0, sg1, ss0, ss1):
  c = lax.axis_index("c")
  s = lax.axis_index("s")
  gb, jb, rb, sg, ss = (g0, g1), (j0, j1), (r0, r1), (sg0, sg1), (ss0, ss1)
  dummy = xflat_hbm.at[pl.ds(0, KG * BE), :]

  @pl.loop(0, CPS)
  def _(ci):
    ch = c * CPS + ci
    _zero_acc(zeros_hbm, acc, s)
    plsc.subcore_barrier()

    def issue(bi, g):
      base = s * EPT + g * (KG * BE)
      pltpu.sync_copy(gidx_hbm.at[ch, pl.ds(base, KG * BE)], gb[bi])
      pltpu.sync_copy(jidx_hbm.at[pl.ds(base, KG * BE)], jb[bi])
      pltpu.async_copy(xflat_hbm.at[gb[bi]], rb[bi], sg[bi])

    def gwait(bi):
      pltpu.make_async_copy(dummy, rb[bi], sg[bi]).wait()

    def sstart(bi):
      pltpu.async_copy(rb[bi], acc.at[jb[bi]], ss[bi], add=True)

    def swait(bi):
      pltpu.make_async_copy(dummy, rb[bi], ss[bi]).wait()

    issue(0, 0)

    @pl.loop(0, NG // 2)
    def _(p):
      @pl.when(p > 0)
      def _():
        swait(1)
      issue(1, 2 * p + 1)
      gwait(0)
      sstart(0)

      @pl.when(p < NG // 2 - 1)
      def _():
        swait(0)
        issue(0, 2 * p + 2)
      gwait(1)
      sstart(1)

    swait(0)
    swait(1)
    plsc.subcore_barrier()
    pltpu.sync_copy(acc.at[pl.ds(s * WB, WB), :],
                    s_hbm.at[pl.ds(s * WB, WB), pl.ds(ch * LANES, LANES)])
    plsc.subcore_barrier()


def _combine_body(relu, s_ref, inv_ref, x_ref, w_ref, root_ref, b_ref, o_ref):
  acc = jnp.dot(x_ref[...], root_ref[...], preferred_element_type=jnp.float32)
  acc = acc + b_ref[...]
  for r in range(R):
    sr = s_ref[r] * inv_ref[:, r:r + 1]
    acc = acc + jnp.dot(sr, w_ref[r], preferred_element_type=jnp.float32)
  if relu:
    acc = jnp.maximum(acc, 0.0)
  o_ref[...] = acc


def _combine(s, inv_t, x, w, root, bias, relu):
  return pl.pallas_call(
      functools.partial(_combine_body, relu),
      out_shape=jax.ShapeDtypeStruct((N, H), jnp.float32),
      grid=(N // TN,),
      in_specs=[
          pl.BlockSpec((R, TN, H), lambda i: (0, i, 0)),
          pl.BlockSpec((TN, R), lambda i: (i, 0)),
          pl.BlockSpec((TN, H), lambda i: (i, 0)),
          pl.BlockSpec((R, H, H), lambda i: (0, 0, 0)),
          pl.BlockSpec((H, H), lambda i: (0, 0)),
          pl.BlockSpec((1, H), lambda i: (0, 0)),
      ],
      out_specs=pl.BlockSpec((TN, H), lambda i: (i, 0)),
  )(s.reshape(R, N, H), inv_t, x, w, root, bias.reshape(1, H))


def _expand_blockdiag(w):
  nb, bi, bo = w.shape[1], w.shape[2], w.shape[3]
  wd = jnp.zeros((R, nb * bi, nb * bo), jnp.float32)
  for b in range(nb):
    wd = wd.at[:, b * bi:(b + 1) * bi, b * bo:(b + 1) * bo].set(w[:, b])
  return wd


def kernel(x, edge_index, edge_type, w0, root0, b0, w1, root1, b1, w2, root2,
           b2):
  src = edge_index[0].astype(jnp.int32)
  dst = edge_index[1].astype(jnp.int32)
  et = edge_type.astype(jnp.int32)
  pad = E_PAD - E
  gsrc = jnp.concatenate([src, jnp.zeros((pad,), jnp.int32)])
  gidx = (gsrc[None, :] * NCH +
          jnp.arange(NCH, dtype=jnp.int32)[:, None])
  jidx = jnp.concatenate([et * N + dst, jnp.full((pad,), ROWS, jnp.int32)])
  zeros = jnp.zeros((ZROWS, LANES), jnp.float32)
  ones = jnp.ones((KG * BE, LANES), jnp.float32)

  cnt = _sc_counts(jidx, zeros, ones)
  csum = cnt[0, :, 0] + cnt[1, :, 0]
  inv_t = (1.0 / jnp.clip(csum, 1.0)).reshape(R, N).T

  wd0 = _expand_blockdiag(w0)
  wd1 = _expand_blockdiag(w1)

  h = x
  for (w, root, b, relu) in ((wd0, root0, b0, True), (wd1, root1, b1, True),
                             (w2, root2, b2, False)):
    s = _sc_scatter(h.reshape(N * NCH, LANES), gidx, jidx, zeros)
    h = _combine(s, inv_t, h, w, root, b, relu)
  return h
